# Initial kernel scaffold; baseline (speedup 1.0000x reference)
import jax
import jax.numpy as jnp
from jax.experimental import pallas as pl


def kernel(token_id, table):
    raise NotImplementedError("write your pallas kernel here")


# speedup vs baseline: 1.2689x; 1.2689x over previous
import functools
import math

import jax
import jax.numpy as jnp
import numpy as np
from jax import lax
from jax.experimental import pallas as pl
from jax.experimental.pallas import tpu as pltpu
from jax.experimental.pallas import tpu_sc as plsc

VOCAB = 50267
EMBED_DIM = 128
SEQ_LEN = 2048


def _positional_encoding_np(seq_len: int, d: int) -> np.ndarray:
    position = np.arange(seq_len, dtype=np.float32)[:, None]
    div_term = np.exp(
        np.arange(0, d, 2, dtype=np.float32) * (-math.log(10000.0) / d)
    ).astype(np.float32)
    pe = np.zeros((seq_len, d), dtype=np.float32)
    pe[:, 0::2] = np.sin(position * div_term)
    pe[:, 1::2] = np.cos(position * div_term)
    return pe


_PE = _positional_encoding_np(SEQ_LEN, EMBED_DIM)


@functools.lru_cache(maxsize=None)
def _build_sc_kernel():
    info = plsc.get_sparse_core_info()
    NC, NS, L = info.num_cores, info.num_subcores, info.num_lanes
    NW = NC * NS
    BPW = SEQ_LEN // NW

    mesh = plsc.VectorSubcoreMesh(core_axis_name="c", subcore_axis_name="s")

    @functools.partial(
        pl.kernel,
        mesh=mesh,
        out_type=jax.ShapeDtypeStruct((SEQ_LEN, EMBED_DIM), jnp.float32),
        scratch_types=[
            pltpu.VMEM((BPW,), jnp.int32),
            pltpu.VMEM((BPW, EMBED_DIM), jnp.float32),
            pltpu.VMEM((BPW, EMBED_DIM), jnp.float32),
            pltpu.SemaphoreType.DMA,
        ],
    )
    def emb_kernel(idx_hbm, table_hbm, pe_hbm, out_hbm, idx_v, rows_v, pe_v, sem):
        wid = lax.axis_index("s") * NC + lax.axis_index("c")
        base = wid * BPW
        pltpu.sync_copy(idx_hbm.at[pl.ds(base, BPW)], idx_v)
        gather = pltpu.async_copy(table_hbm.at[idx_v], rows_v, sem)
        pltpu.sync_copy(pe_hbm.at[pl.ds(base, BPW)], pe_v)
        gather.wait()

        def add_row(r, carry):
            for c in range(EMBED_DIM // L):
                sl = pl.ds(c * L, L)
                rows_v[r, sl] = rows_v[r, sl] + pe_v[r, sl]
            return carry

        lax.fori_loop(0, BPW, add_row, 0)
        pltpu.sync_copy(rows_v, out_hbm.at[pl.ds(base, BPW)])

    return emb_kernel


def kernel(token_id, table):
    emb = _build_sc_kernel()
    pe = jnp.asarray(_PE)
    out = emb(token_id.astype(jnp.int32), table, pe)
    return out[None, :, :]


# speedup vs baseline: 1.2691x; 1.0002x over previous
import functools
import math

import jax
import jax.numpy as jnp
import numpy as np
from jax import lax
from jax.experimental import pallas as pl
from jax.experimental.pallas import tpu as pltpu
from jax.experimental.pallas import tpu_sc as plsc

VOCAB = 50267
EMBED_DIM = 128
SEQ_LEN = 2048


def _positional_encoding_np(seq_len: int, d: int) -> np.ndarray:
    position = np.arange(seq_len, dtype=np.float32)[:, None]
    div_term = np.exp(
        np.arange(0, d, 2, dtype=np.float32) * (-math.log(10000.0) / d)
    ).astype(np.float32)
    pe = np.zeros((seq_len, d), dtype=np.float32)
    pe[:, 0::2] = np.sin(position * div_term)
    pe[:, 1::2] = np.cos(position * div_term)
    return pe


_PE = _positional_encoding_np(SEQ_LEN, EMBED_DIM)


@functools.lru_cache(maxsize=None)
def _build_sc_kernel():
    info = plsc.get_sparse_core_info()
    NC, NS, L = info.num_cores, info.num_subcores, info.num_lanes
    NW = NC * NS
    BPW = SEQ_LEN // NW

    mesh = plsc.VectorSubcoreMesh(core_axis_name="c", subcore_axis_name="s")

    @functools.partial(
        pl.kernel,
        mesh=mesh,
        out_type=jax.ShapeDtypeStruct((SEQ_LEN, EMBED_DIM), jnp.float32),
        scratch_types=[
            pltpu.VMEM((BPW,), jnp.int32),
            pltpu.VMEM((BPW, EMBED_DIM), jnp.float32),
            pltpu.SemaphoreType.DMA,
        ],
    )
    def emb_kernel(idx_hbm, table_hbm, pe_hbm, out_hbm, idx_v, rows_v, sem):
        wid = lax.axis_index("s") * NC + lax.axis_index("c")
        base = wid * BPW
        pltpu.sync_copy(idx_hbm.at[pl.ds(base, BPW)], idx_v)
        pltpu.sync_copy(pe_hbm.at[pl.ds(base, BPW)], rows_v)
        pltpu.async_copy(table_hbm.at[idx_v], rows_v, sem, add=True).wait()
        pltpu.sync_copy(rows_v, out_hbm.at[pl.ds(base, BPW)])

    return emb_kernel


def kernel(token_id, table):
    emb = _build_sc_kernel()
    pe = jnp.asarray(_PE)
    out = emb(token_id.astype(jnp.int32), table, pe)
    return out[None, :, :]


# speedup vs baseline: 1.2885x; 1.0153x over previous
import functools
import math

import jax
import jax.numpy as jnp
import numpy as np
from jax import lax
from jax.experimental import pallas as pl
from jax.experimental.pallas import tpu as pltpu
from jax.experimental.pallas import tpu_sc as plsc

VOCAB = 50267
EMBED_DIM = 128
SEQ_LEN = 2048


def _positional_encoding_np(seq_len: int, d: int) -> np.ndarray:
    position = np.arange(seq_len, dtype=np.float32)[:, None]
    div_term = np.exp(
        np.arange(0, d, 2, dtype=np.float32) * (-math.log(10000.0) / d)
    ).astype(np.float32)
    pe = np.zeros((seq_len, d), dtype=np.float32)
    pe[:, 0::2] = np.sin(position * div_term)
    pe[:, 1::2] = np.cos(position * div_term)
    return pe


_PE = _positional_encoding_np(SEQ_LEN, EMBED_DIM)


@functools.lru_cache(maxsize=None)
def _build_sc_kernel():
    info = plsc.get_sparse_core_info()
    NC, NS, L = info.num_cores, info.num_subcores, info.num_lanes
    NW = NC * NS
    BPW = SEQ_LEN // NW

    mesh = plsc.VectorSubcoreMesh(core_axis_name="c", subcore_axis_name="s")

    @functools.partial(
        pl.kernel,
        mesh=mesh,
        out_type=jax.ShapeDtypeStruct((SEQ_LEN, EMBED_DIM), jnp.float32),
        scratch_types=[
            pltpu.VMEM((BPW,), jnp.int32),
            pltpu.VMEM((BPW, EMBED_DIM), jnp.float32),
            pltpu.SemaphoreType.DMA,
            pltpu.SemaphoreType.DMA,
            pltpu.SemaphoreType.DMA,
            pltpu.SemaphoreType.DMA,
            pltpu.SemaphoreType.DMA,
        ],
    )
    def emb_kernel(idx_hbm, table_hbm, pe_hbm, out_hbm, idx_v, rows_v,
                   sem_i, sem_p, sem_g0, sem_g1, sem_o0):
        wid = lax.axis_index("s") * NC + lax.axis_index("c")
        base = wid * BPW
        H = BPW // 2
        ci = pltpu.async_copy(idx_hbm.at[pl.ds(base, BPW)], idx_v, sem_i)
        cp = pltpu.async_copy(pe_hbm.at[pl.ds(base, BPW)], rows_v, sem_p)
        ci.wait()
        cp.wait()
        g0 = pltpu.async_copy(
            table_hbm.at[idx_v.at[pl.ds(0, H)]],
            rows_v.at[pl.ds(0, H)], sem_g0, add=True)
        g1 = pltpu.async_copy(
            table_hbm.at[idx_v.at[pl.ds(H, H)]],
            rows_v.at[pl.ds(H, H)], sem_g1, add=True)
        g0.wait()
        o0 = pltpu.async_copy(
            rows_v.at[pl.ds(0, H)], out_hbm.at[pl.ds(base, H)], sem_o0)
        g1.wait()
        pltpu.sync_copy(rows_v.at[pl.ds(H, H)], out_hbm.at[pl.ds(base + H, H)])
        o0.wait()

    return emb_kernel


def kernel(token_id, table):
    emb = _build_sc_kernel()
    pe = jnp.asarray(_PE)
    out = emb(token_id.astype(jnp.int32), table, pe)
    return out[None, :, :]


# speedup vs baseline: 1.3002x; 1.0091x over previous
import functools
import math

import jax
import jax.numpy as jnp
import numpy as np
from jax import lax
from jax.experimental import pallas as pl
from jax.experimental.pallas import tpu as pltpu
from jax.experimental.pallas import tpu_sc as plsc

VOCAB = 50267
EMBED_DIM = 128
SEQ_LEN = 2048


def _positional_encoding_np(seq_len: int, d: int) -> np.ndarray:
    position = np.arange(seq_len, dtype=np.float32)[:, None]
    div_term = np.exp(
        np.arange(0, d, 2, dtype=np.float32) * (-math.log(10000.0) / d)
    ).astype(np.float32)
    pe = np.zeros((seq_len, d), dtype=np.float32)
    pe[:, 0::2] = np.sin(position * div_term)
    pe[:, 1::2] = np.cos(position * div_term)
    return pe


_PE = _positional_encoding_np(SEQ_LEN, EMBED_DIM)

_NCHUNK = 4


@functools.lru_cache(maxsize=None)
def _build_sc_kernel():
    info = plsc.get_sparse_core_info()
    NC, NS, L = info.num_cores, info.num_subcores, info.num_lanes
    NW = NC * NS
    BPW = SEQ_LEN // NW

    mesh = plsc.VectorSubcoreMesh(core_axis_name="c", subcore_axis_name="s")

    @functools.partial(
        pl.kernel,
        mesh=mesh,
        out_type=jax.ShapeDtypeStruct((SEQ_LEN, EMBED_DIM), jnp.float32),
        scratch_types=[
            pltpu.VMEM((BPW,), jnp.int32),
            pltpu.VMEM((BPW, EMBED_DIM), jnp.float32),
            pltpu.SemaphoreType.DMA,
        ]
        + [pltpu.SemaphoreType.DMA] * (3 * _NCHUNK),
    )
    def emb_kernel(idx_hbm, table_hbm, pe_hbm, out_hbm, idx_v, rows_v,
                   sem_i, *sems):
        sem_p = sems[:_NCHUNK]
        sem_g = sems[_NCHUNK:2 * _NCHUNK]
        sem_o = sems[2 * _NCHUNK:]
        wid = lax.axis_index("s") * NC + lax.axis_index("c")
        base = wid * BPW
        H = BPW // _NCHUNK
        ci = pltpu.async_copy(idx_hbm.at[pl.ds(base, BPW)], idx_v, sem_i)
        cps = [
            pltpu.async_copy(pe_hbm.at[pl.ds(base + c * H, H)],
                             rows_v.at[pl.ds(c * H, H)], sem_p[c])
            for c in range(_NCHUNK)
        ]
        ci.wait()
        gs = []
        for c in range(_NCHUNK):
            cps[c].wait()
            gs.append(pltpu.async_copy(
                table_hbm.at[idx_v.at[pl.ds(c * H, H)]],
                rows_v.at[pl.ds(c * H, H)], sem_g[c], add=True))
        os_ = []
        for c in range(_NCHUNK):
            gs[c].wait()
            os_.append(pltpu.async_copy(
                rows_v.at[pl.ds(c * H, H)],
                out_hbm.at[pl.ds(base + c * H, H)], sem_o[c]))
        for c in range(_NCHUNK):
            os_[c].wait()

    return emb_kernel


def kernel(token_id, table):
    emb = _build_sc_kernel()
    pe = jnp.asarray(_PE)
    out = emb(token_id.astype(jnp.int32), table, pe)
    return out[None, :, :]


# speedup vs baseline: 1.3314x; 1.0240x over previous
import functools
import math

import jax
import jax.numpy as jnp
import numpy as np
from jax import lax
from jax.experimental import pallas as pl
from jax.experimental.pallas import tpu as pltpu
from jax.experimental.pallas import tpu_sc as plsc

VOCAB = 50267
EMBED_DIM = 128
SEQ_LEN = 2048


def _positional_encoding_np(seq_len: int, d: int) -> np.ndarray:
    position = np.arange(seq_len, dtype=np.float32)[:, None]
    div_term = np.exp(
        np.arange(0, d, 2, dtype=np.float32) * (-math.log(10000.0) / d)
    ).astype(np.float32)
    pe = np.zeros((seq_len, d), dtype=np.float32)
    pe[:, 0::2] = np.sin(position * div_term)
    pe[:, 1::2] = np.cos(position * div_term)
    return pe


_PE = _positional_encoding_np(SEQ_LEN, EMBED_DIM)

_NCHUNK = 4


@functools.lru_cache(maxsize=None)
def _build_sc_kernel():
    info = plsc.get_sparse_core_info()
    NC, NS, L = 1, info.num_subcores, info.num_lanes
    NW = NC * NS
    BPW = SEQ_LEN // NW

    mesh = plsc.VectorSubcoreMesh(
        core_axis_name="c", subcore_axis_name="s", num_cores=NC)

    @functools.partial(
        pl.kernel,
        mesh=mesh,
        out_type=jax.ShapeDtypeStruct((SEQ_LEN, EMBED_DIM), jnp.float32),
        scratch_types=[
            pltpu.VMEM((BPW,), jnp.int32),
            pltpu.VMEM((BPW, EMBED_DIM), jnp.float32),
            pltpu.SemaphoreType.DMA,
        ]
        + [pltpu.SemaphoreType.DMA] * (3 * _NCHUNK),
    )
    def emb_kernel(idx_hbm, table_hbm, pe_hbm, out_hbm, idx_v, rows_v,
                   sem_i, *sems):
        sem_p = sems[:_NCHUNK]
        sem_g = sems[_NCHUNK:2 * _NCHUNK]
        sem_o = sems[2 * _NCHUNK:]
        wid = lax.axis_index("s") * NC + lax.axis_index("c")
        base = wid * BPW
        H = BPW // _NCHUNK
        ci = pltpu.async_copy(idx_hbm.at[pl.ds(base, BPW)], idx_v, sem_i)
        cps = [
            pltpu.async_copy(pe_hbm.at[pl.ds(base + c * H, H)],
                             rows_v.at[pl.ds(c * H, H)], sem_p[c])
            for c in range(_NCHUNK)
        ]
        ci.wait()
        gs = []
        for c in range(_NCHUNK):
            cps[c].wait()
            gs.append(pltpu.async_copy(
                table_hbm.at[idx_v.at[pl.ds(c * H, H)]],
                rows_v.at[pl.ds(c * H, H)], sem_g[c], add=True))
        os_ = []
        for c in range(_NCHUNK):
            gs[c].wait()
            os_.append(pltpu.async_copy(
                rows_v.at[pl.ds(c * H, H)],
                out_hbm.at[pl.ds(base + c * H, H)], sem_o[c]))
        for c in range(_NCHUNK):
            os_[c].wait()

    return emb_kernel


def kernel(token_id, table):
    emb = _build_sc_kernel()
    pe = jnp.asarray(_PE)
    out = emb(token_id.astype(jnp.int32), table, pe)
    return out[None, :, :]


# speedup vs baseline: 1.3453x; 1.0105x over previous
import functools
import math

import jax
import jax.numpy as jnp
import numpy as np
from jax import lax
from jax.experimental import pallas as pl
from jax.experimental.pallas import tpu as pltpu
from jax.experimental.pallas import tpu_sc as plsc

VOCAB = 50267
EMBED_DIM = 128
SEQ_LEN = 2048


def _positional_encoding_np(seq_len: int, d: int) -> np.ndarray:
    position = np.arange(seq_len, dtype=np.float32)[:, None]
    div_term = np.exp(
        np.arange(0, d, 2, dtype=np.float32) * (-math.log(10000.0) / d)
    ).astype(np.float32)
    pe = np.zeros((seq_len, d), dtype=np.float32)
    pe[:, 0::2] = np.sin(position * div_term)
    pe[:, 1::2] = np.cos(position * div_term)
    return pe


_PE = _positional_encoding_np(SEQ_LEN, EMBED_DIM)

_NCHUNK = 8


@functools.lru_cache(maxsize=None)
def _build_sc_kernel():
    info = plsc.get_sparse_core_info()
    NC, NS, L = 1, info.num_subcores, info.num_lanes
    NW = NC * NS
    BPW = SEQ_LEN // NW

    mesh = plsc.VectorSubcoreMesh(
        core_axis_name="c", subcore_axis_name="s", num_cores=NC)

    @functools.partial(
        pl.kernel,
        mesh=mesh,
        out_type=jax.ShapeDtypeStruct((SEQ_LEN, EMBED_DIM), jnp.float32),
        scratch_types=[
            pltpu.VMEM((BPW,), jnp.int32),
            pltpu.VMEM((BPW, EMBED_DIM), jnp.float32),
            pltpu.SemaphoreType.DMA,
        ]
        + [pltpu.SemaphoreType.DMA] * (3 * _NCHUNK),
    )
    def emb_kernel(idx_hbm, table_hbm, pe_hbm, out_hbm, idx_v, rows_v,
                   sem_i, *sems):
        sem_p = sems[:_NCHUNK]
        sem_g = sems[_NCHUNK:2 * _NCHUNK]
        sem_o = sems[2 * _NCHUNK:]
        wid = lax.axis_index("s") * NC + lax.axis_index("c")
        base = wid * BPW
        H = BPW // _NCHUNK
        ci = pltpu.async_copy(idx_hbm.at[pl.ds(base, BPW)], idx_v, sem_i)
        cps = [
            pltpu.async_copy(pe_hbm.at[pl.ds(base + c * H, H)],
                             rows_v.at[pl.ds(c * H, H)], sem_p[c])
            for c in range(_NCHUNK)
        ]
        ci.wait()
        gs = []
        for c in range(_NCHUNK):
            cps[c].wait()
            gs.append(pltpu.async_copy(
                table_hbm.at[idx_v.at[pl.ds(c * H, H)]],
                rows_v.at[pl.ds(c * H, H)], sem_g[c], add=True))
        os_ = []
        for c in range(_NCHUNK):
            gs[c].wait()
            os_.append(pltpu.async_copy(
                rows_v.at[pl.ds(c * H, H)],
                out_hbm.at[pl.ds(base + c * H, H)], sem_o[c]))
        for c in range(_NCHUNK):
            os_[c].wait()

    return emb_kernel


def kernel(token_id, table):
    emb = _build_sc_kernel()
    pe = jnp.asarray(_PE)
    out = emb(token_id.astype(jnp.int32), table, pe)
    return out[None, :, :]
